# baseline (speedup 1.0000x reference)
import functools

import jax
import jax.numpy as jnp
from jax import lax
from jax.experimental import pallas as pl
from jax.experimental.pallas import tpu as pltpu
from jax.experimental.pallas import tpu_sc as plsc

B = 4096
F = 26
V = 100000
D = 64
DENSE = 13
H1 = 1024
H2 = 512

N_ROWS = B * F
CHUNK = 128


_NC = 2
_NS = 16


def _make_gather():
    nw = _NC * _NS
    rows_per_w = N_ROWS // nw
    n_chunks = rows_per_w // CHUNK
    mesh = plsc.VectorSubcoreMesh(
        core_axis_name="c", subcore_axis_name="s",
        num_cores=_NC, num_subcores=_NS,
    )

    @functools.partial(
        pl.kernel,
        mesh=mesh,
        out_type=jax.ShapeDtypeStruct((N_ROWS, D), jnp.float32),
        scratch_types=[
            pltpu.VMEM((n_chunks, CHUNK), jnp.int32),
            pltpu.VMEM((CHUNK, D), jnp.float32),
            pltpu.VMEM((CHUNK, D), jnp.float32),
            pltpu.SemaphoreType.DMA,
            pltpu.SemaphoreType.DMA,
        ],
        compiler_params=pltpu.CompilerParams(use_tc_tiling_on_sc=False),
    )
    def gather_k(table_hbm, ids_hbm, out_hbm, idx_v, rows0, rows1, sem0, sem1):
        wid = lax.axis_index("s") * _NC + lax.axis_index("c")
        base = wid * rows_per_w
        pltpu.sync_copy(ids_hbm.at[wid], idx_v)

        bufs = (rows0, rows1)
        sems = (sem0, sem1)

        first = pltpu.make_async_copy(table_hbm.at[idx_v.at[0]], bufs[0], sems[0])
        first.start()

        def body(j, _):
            slot = lax.rem(j, 2)

            def do(s):
                nxt = (s + 1) % 2

                @pl.when(j + 1 < n_chunks)
                def _():
                    pltpu.make_async_copy(
                        table_hbm.at[idx_v.at[j + 1]], bufs[nxt], sems[nxt]
                    ).start()

                pltpu.make_async_copy(
                    table_hbm.at[idx_v.at[j]], bufs[s], sems[s]
                ).wait()
                pltpu.sync_copy(bufs[s], out_hbm.at[pl.ds(base + j * CHUNK, CHUNK)])

            @pl.when(slot == 0)
            def _():
                do(0)

            @pl.when(slot == 1)
            def _():
                do(1)

            return 0

        lax.fori_loop(0, n_chunks, body, 0)

    return gather_k, nw, rows_per_w


_gather_kernel, _NW, _ROWS_PER_W = _make_gather()


def _mlp_body(x_ref, d_ref, w1_ref, w1d_ref, b1_ref, w2_ref, b2_ref,
              w3_ref, b3_ref, o_ref):
    h = jnp.maximum(
        jnp.dot(x_ref[...], w1_ref[...], preferred_element_type=jnp.float32)
        + jnp.dot(d_ref[...], w1d_ref[...], preferred_element_type=jnp.float32)
        + b1_ref[...],
        0.0,
    )
    h = jnp.maximum(
        jnp.dot(h, w2_ref[...], preferred_element_type=jnp.float32) + b2_ref[...],
        0.0,
    )
    logit = jnp.dot(h, w3_ref[...], preferred_element_type=jnp.float32) + b3_ref[...]
    o_ref[...] = jax.nn.sigmoid(logit)


_BM = 512


def _mlp(x, dpad, w1, w1d, b1, w2, b2, w3, b3):
    in_dim = F * D
    grid = (B // _BM,)
    return pl.pallas_call(
        _mlp_body,
        grid=grid,
        in_specs=[
            pl.BlockSpec((_BM, in_dim), lambda i: (i, 0)),
            pl.BlockSpec((_BM, 128), lambda i: (i, 0)),
            pl.BlockSpec((in_dim, H1), lambda i: (0, 0)),
            pl.BlockSpec((128, H1), lambda i: (0, 0)),
            pl.BlockSpec((1, H1), lambda i: (0, 0)),
            pl.BlockSpec((H1, H2), lambda i: (0, 0)),
            pl.BlockSpec((1, H2), lambda i: (0, 0)),
            pl.BlockSpec((H2, 1), lambda i: (0, 0)),
            pl.BlockSpec((1, 1), lambda i: (0, 0)),
        ],
        out_specs=pl.BlockSpec((_BM, 1), lambda i: (i, 0)),
        out_shape=jax.ShapeDtypeStruct((B, 1), jnp.float32),
        compiler_params=pltpu.CompilerParams(
            dimension_semantics=("arbitrary",),
        ),
    )(x, dpad, w1, w1d, b1, w2, b2, w3, b3)


def kernel(sparse_ids, dense_feats, tables, W1, b1, W2, b2, W3, b3):
    flat_table = tables.reshape(F * V, D)
    flat_ids = (sparse_ids.astype(jnp.int32)
                + (jnp.arange(F, dtype=jnp.int32) * V)[None, :])
    ids3 = flat_ids.reshape(_NW, _ROWS_PER_W // CHUNK, CHUNK)

    emb = _gather_kernel(flat_table, ids3)
    x = emb.reshape(B, F * D)

    dpad = jnp.pad(dense_feats, ((0, 0), (0, 128 - DENSE)))
    w1a = W1[: F * D]
    w1d = jnp.pad(W1[F * D:], ((0, 128 - DENSE), (0, 0)))

    return _mlp(x, dpad, w1a, w1d, b1.reshape(1, H1), W2, b2.reshape(1, H2),
                W3, b3.reshape(1, 1))
